# baseline (speedup 1.0000x reference)
import functools

import jax
import jax.numpy as jnp
from jax import lax
from jax.experimental import pallas as pl
from jax.experimental.pallas import tpu as pltpu
from jax.experimental.pallas import tpu_sc as plsc

N = 10000
NP = 10240
F = 128
E = 320000
NC = 2
NS = 16
NW = NC * NS
CHUNK = 128
NCHUNK = 80
EP = NW * NCHUNK * CHUNK
STRIPE = NP // NS

_mesh = plsc.VectorSubcoreMesh(core_axis_name="c", subcore_axis_name="s")


@functools.partial(
    pl.kernel,
    out_type=jax.ShapeDtypeStruct((NC, NP, F), jnp.float32),
    mesh=_mesh,
    scratch_types=[
        pltpu.VMEM((2, NCHUNK, CHUNK), jnp.int32),
        pltpu.VMEM((CHUNK, F), jnp.float32),
        pltpu.VMEM_SHARED((NP, F), jnp.float32),
    ],
)
def _deg_kernel(src_hbm, dst_hbm, ones_hbm, zeros128_hbm,
                deg_out, idx_v, ones_v, tab_sh):
    c = lax.axis_index("c")
    s = lax.axis_index("s")
    pltpu.sync_copy(zeros128_hbm, tab_sh.at[pl.ds(s * STRIPE, STRIPE)])
    pltpu.sync_copy(ones_hbm, ones_v)

    @pl.when(c == 0)
    def _():
        pltpu.sync_copy(src_hbm.at[pl.ds(2 * s, 2)], idx_v)

    @pl.when(c == 1)
    def _():
        pltpu.sync_copy(dst_hbm.at[pl.ds(2 * s, 2)], idx_v)

    plsc.subcore_barrier()

    def body(j, carry):
        pltpu.sync_copy(ones_v, tab_sh.at[idx_v.at[0, j]], add=True)
        pltpu.sync_copy(ones_v, tab_sh.at[idx_v.at[1, j]], add=True)
        return carry

    lax.fori_loop(0, NCHUNK, body, 0)
    plsc.subcore_barrier()
    pltpu.sync_copy(tab_sh.at[pl.ds(s * STRIPE, STRIPE)],
                    deg_out.at[c, pl.ds(s * STRIPE, STRIPE)])


@functools.partial(
    pl.kernel,
    out_type=jax.ShapeDtypeStruct((NC, NP, F), jnp.float32),
    mesh=_mesh,
    scratch_types=[
        pltpu.VMEM((NCHUNK // 2, CHUNK), jnp.int32),
        pltpu.VMEM((NCHUNK // 2, CHUNK), jnp.int32),
        pltpu.VMEM((2, CHUNK, F), jnp.float32),
        pltpu.VMEM_SHARED((NP, F), jnp.float32),
        pltpu.SemaphoreType.DMA((2,)),
    ],
)
def _mp_kernel(table_hbm, src_hbm, dst_hbm, zeros128_hbm,
               agg_out,
               src_v, dst_v, rows, acc_sh, sem):
    c = lax.axis_index("c")
    s = lax.axis_index("s")
    wid = s * NC + c
    H = NCHUNK // 2
    pltpu.sync_copy(zeros128_hbm, acc_sh.at[pl.ds(s * STRIPE, STRIPE)])
    plsc.subcore_barrier()

    for h in range(2):
        pltpu.sync_copy(src_hbm.at[wid, pl.ds(h * H, H)], src_v)
        pltpu.sync_copy(dst_hbm.at[wid, pl.ds(h * H, H)], dst_v)

        def body(j, carry):
            pltpu.async_copy(table_hbm.at[src_v.at[j]], rows.at[0],
                             sem.at[0]).wait()
            pltpu.sync_copy(rows.at[0], acc_sh.at[dst_v.at[j]], add=True)
            return carry

        lax.fori_loop(0, H, body, 0)
    plsc.subcore_barrier()
    pltpu.sync_copy(acc_sh.at[pl.ds(s * STRIPE, STRIPE)],
                    agg_out.at[c, pl.ds(s * STRIPE, STRIPE)])


BR = 512


def _norm_body(d0_ref, d1_ref, ns_ref, nd_ref):
    dsrc = d0_ref[...]
    ddst = d1_ref[...]
    ns_ref[...] = lax.rsqrt(jnp.maximum(dsrc, 1.0))
    nd_ref[...] = lax.rsqrt(jnp.maximum(ddst, 1.0))


def _norms(dsrc, ddst):
    return pl.pallas_call(
        _norm_body,
        out_shape=(
            jax.ShapeDtypeStruct((NP, 1), jnp.float32),
            jax.ShapeDtypeStruct((NP, 1), jnp.float32),
        ),
    )(dsrc, ddst)


def _first_body(x_ref, w_ref, ns_ref, o_ref):
    o_ref[...] = jnp.dot(x_ref[...], w_ref[...],
                         preferred_element_type=jnp.float32) * ns_ref[...]


def _first_table(x, w, ns):
    return pl.pallas_call(
        _first_body,
        grid=(NP // BR,),
        in_specs=[
            pl.BlockSpec((BR, F), lambda i: (i, 0)),
            pl.BlockSpec((F, F), lambda i: (0, 0)),
            pl.BlockSpec((BR, 1), lambda i: (i, 0)),
        ],
        out_specs=pl.BlockSpec((BR, F), lambda i: (i, 0)),
        out_shape=jax.ShapeDtypeStruct((NP, F), jnp.float32),
    )(x, w, ns)


def _mid_body(a_ref, nd_ref, b_ref, w_ref, ns_ref, o_ref):
    h = (a_ref[0] + a_ref[1]) * nd_ref[...] + b_ref[...]
    h = jnp.maximum(h, 0.0)
    o_ref[...] = jnp.dot(h, w_ref[...],
                         preferred_element_type=jnp.float32) * ns_ref[...]


def _mid_table(agg, nd, b, w, ns):
    return pl.pallas_call(
        _mid_body,
        grid=(NP // BR,),
        in_specs=[
            pl.BlockSpec((NC, BR, F), lambda i: (0, i, 0)),
            pl.BlockSpec((BR, 1), lambda i: (i, 0)),
            pl.BlockSpec((1, F), lambda i: (0, 0)),
            pl.BlockSpec((F, F), lambda i: (0, 0)),
            pl.BlockSpec((BR, 1), lambda i: (i, 0)),
        ],
        out_specs=pl.BlockSpec((BR, F), lambda i: (i, 0)),
        out_shape=jax.ShapeDtypeStruct((NP, F), jnp.float32),
    )(agg, nd, b, w, ns)


def _final_body(a_ref, nd_ref, b_ref, o_ref):
    o_ref[...] = (a_ref[0] + a_ref[1]) * nd_ref[...] + b_ref[...]


def _final(agg, nd, b):
    return pl.pallas_call(
        _final_body,
        grid=(NP // BR,),
        in_specs=[
            pl.BlockSpec((NC, BR, F), lambda i: (0, i, 0)),
            pl.BlockSpec((BR, 1), lambda i: (i, 0)),
            pl.BlockSpec((1, F), lambda i: (0, 0)),
        ],
        out_specs=pl.BlockSpec((BR, F), lambda i: (i, 0)),
        out_shape=jax.ShapeDtypeStruct((NP, F), jnp.float32),
    )(agg, nd, b)


def kernel(features, edge_index, W1, b1, W2, b2, W3, b3):
    src = edge_index[0].astype(jnp.int32)
    dst = edge_index[1].astype(jnp.int32)
    padv = jnp.full((EP - E,), N, dtype=jnp.int32)
    src_p = jnp.concatenate([src, padv]).reshape(NW, NCHUNK, CHUNK)
    dst_p = jnp.concatenate([dst, padv]).reshape(NW, NCHUNK, CHUNK)
    x = jnp.zeros((NP, F), jnp.float32).at[:N].set(features)
    w3p = jnp.zeros((F, F), jnp.float32).at[:, : b3.shape[0]].set(W3)
    b3p = jnp.zeros((1, F), jnp.float32).at[0, : b3.shape[0]].set(b3)
    ones128 = jnp.ones((CHUNK, F), jnp.float32)
    zeros128 = jnp.zeros((STRIPE, F), jnp.float32)

    deg = _deg_kernel(src_p, dst_p, ones128, zeros128)
    dsrc = deg[0, :, :1]
    ddst = deg[1, :, :1]
    ns, nd = _norms(dsrc, ddst)

    t1 = _first_table(x, W1, ns)
    agg1 = _mp_kernel(t1, src_p, dst_p, zeros128)
    t2 = _mid_table(agg1, nd, b1.reshape(1, F), W2, ns)
    agg2 = _mp_kernel(t2, src_p, dst_p, zeros128)
    t3 = _mid_table(agg2, nd, b2.reshape(1, F), w3p, ns)
    agg3 = _mp_kernel(t3, src_p, dst_p, zeros128)

    out = _final(agg3, nd, b3p)
    return out[:N, : b3.shape[0]]


# speedup vs baseline: 1.3960x; 1.3960x over previous
import functools

import jax
import jax.numpy as jnp
from jax import lax
from jax.experimental import pallas as pl
from jax.experimental.pallas import tpu as pltpu
from jax.experimental.pallas import tpu_sc as plsc

N = 10000
NP = 10240
F = 128
E = 320000
NC = 2
NS = 16
NW = NC * NS
CHUNK = 128
NCHUNK = 79
EP = NW * NCHUNK * CHUNK
STRIPE = NP // NS

_mesh = plsc.VectorSubcoreMesh(core_axis_name="c", subcore_axis_name="s")


@functools.partial(
    pl.kernel,
    out_type=jax.ShapeDtypeStruct((NC, NP, F), jnp.float32),
    mesh=_mesh,
    scratch_types=[
        pltpu.VMEM((2, NCHUNK, CHUNK), jnp.int32),
        pltpu.VMEM((CHUNK, F), jnp.float32),
        pltpu.VMEM_SHARED((NP, F), jnp.float32),
    ],
)
def _deg_kernel(src_hbm, dst_hbm, ones_hbm, zeros128_hbm,
                deg_out, idx_v, ones_v, tab_sh):
    c = lax.axis_index("c")
    s = lax.axis_index("s")
    pltpu.sync_copy(zeros128_hbm, tab_sh.at[pl.ds(s * STRIPE, STRIPE)])
    pltpu.sync_copy(ones_hbm, ones_v)

    @pl.when(c == 0)
    def _():
        pltpu.sync_copy(src_hbm.at[pl.ds(2 * s, 2)], idx_v)

    @pl.when(c == 1)
    def _():
        pltpu.sync_copy(dst_hbm.at[pl.ds(2 * s, 2)], idx_v)

    plsc.subcore_barrier()

    def body(j, carry):
        pltpu.sync_copy(ones_v, tab_sh.at[idx_v.at[0, j]], add=True)
        pltpu.sync_copy(ones_v, tab_sh.at[idx_v.at[1, j]], add=True)
        return carry

    lax.fori_loop(0, NCHUNK, body, 0)
    plsc.subcore_barrier()
    pltpu.sync_copy(tab_sh.at[pl.ds(s * STRIPE, STRIPE)],
                    deg_out.at[c, pl.ds(s * STRIPE, STRIPE)])


@functools.partial(
    pl.kernel,
    out_type=jax.ShapeDtypeStruct((NC, NP, F), jnp.float32),
    mesh=_mesh,
    scratch_types=[
        pltpu.VMEM((NCHUNK, CHUNK), jnp.int32),
        pltpu.VMEM((NCHUNK, CHUNK), jnp.int32),
        pltpu.VMEM((CHUNK, F), jnp.float32),
        pltpu.VMEM_SHARED((NP, F), jnp.float32),
        pltpu.SemaphoreType.DMA,
    ],
)
def _mp_kernel(table_hbm, src_hbm, dst_hbm, zeros128_hbm,
               agg_out,
               src_v, dst_v, rows, acc_sh, sem):
    c = lax.axis_index("c")
    s = lax.axis_index("s")
    wid = s * NC + c
    pltpu.sync_copy(zeros128_hbm, acc_sh.at[pl.ds(s * STRIPE, STRIPE)])
    pltpu.sync_copy(src_hbm.at[wid], src_v)
    pltpu.sync_copy(dst_hbm.at[wid], dst_v)
    plsc.subcore_barrier()

    def body(j, carry):
        pltpu.async_copy(table_hbm.at[src_v.at[j]], rows, sem).wait()
        pltpu.sync_copy(rows, acc_sh.at[dst_v.at[j]], add=True)
        return carry

    lax.fori_loop(0, NCHUNK, body, 0)
    plsc.subcore_barrier()
    pltpu.sync_copy(acc_sh.at[pl.ds(s * STRIPE, STRIPE)],
                    agg_out.at[c, pl.ds(s * STRIPE, STRIPE)])


BR = 512


def _norm_body(d0_ref, d1_ref, ns_ref, nd_ref):
    dsrc = d0_ref[...]
    ddst = d1_ref[...]
    ns_ref[...] = lax.rsqrt(jnp.maximum(dsrc, 1.0))
    nd_ref[...] = lax.rsqrt(jnp.maximum(ddst, 1.0))


def _norms(dsrc, ddst):
    return pl.pallas_call(
        _norm_body,
        out_shape=(
            jax.ShapeDtypeStruct((NP, 1), jnp.float32),
            jax.ShapeDtypeStruct((NP, 1), jnp.float32),
        ),
    )(dsrc, ddst)


def _first_body(x_ref, w_ref, ns_ref, o_ref):
    o_ref[...] = jnp.dot(x_ref[...], w_ref[...],
                         preferred_element_type=jnp.float32) * ns_ref[...]


def _first_table(x, w, ns):
    return pl.pallas_call(
        _first_body,
        grid=(NP // BR,),
        in_specs=[
            pl.BlockSpec((BR, F), lambda i: (i, 0)),
            pl.BlockSpec((F, F), lambda i: (0, 0)),
            pl.BlockSpec((BR, 1), lambda i: (i, 0)),
        ],
        out_specs=pl.BlockSpec((BR, F), lambda i: (i, 0)),
        out_shape=jax.ShapeDtypeStruct((NP, F), jnp.float32),
    )(x, w, ns)


def _mid_body(a_ref, nd_ref, b_ref, w_ref, ns_ref, o_ref):
    h = (a_ref[0] + a_ref[1]) * nd_ref[...] + b_ref[...]
    h = jnp.maximum(h, 0.0)
    o_ref[...] = jnp.dot(h, w_ref[...],
                         preferred_element_type=jnp.float32) * ns_ref[...]


def _mid_table(agg, nd, b, w, ns):
    return pl.pallas_call(
        _mid_body,
        grid=(NP // BR,),
        in_specs=[
            pl.BlockSpec((NC, BR, F), lambda i: (0, i, 0)),
            pl.BlockSpec((BR, 1), lambda i: (i, 0)),
            pl.BlockSpec((1, F), lambda i: (0, 0)),
            pl.BlockSpec((F, F), lambda i: (0, 0)),
            pl.BlockSpec((BR, 1), lambda i: (i, 0)),
        ],
        out_specs=pl.BlockSpec((BR, F), lambda i: (i, 0)),
        out_shape=jax.ShapeDtypeStruct((NP, F), jnp.float32),
    )(agg, nd, b, w, ns)


def _final_body(a_ref, nd_ref, b_ref, o_ref):
    o_ref[...] = (a_ref[0] + a_ref[1]) * nd_ref[...] + b_ref[...]


def _final(agg, nd, b):
    return pl.pallas_call(
        _final_body,
        grid=(NP // BR,),
        in_specs=[
            pl.BlockSpec((NC, BR, F), lambda i: (0, i, 0)),
            pl.BlockSpec((BR, 1), lambda i: (i, 0)),
            pl.BlockSpec((1, F), lambda i: (0, 0)),
        ],
        out_specs=pl.BlockSpec((BR, F), lambda i: (i, 0)),
        out_shape=jax.ShapeDtypeStruct((NP, F), jnp.float32),
    )(agg, nd, b)


def kernel(features, edge_index, W1, b1, W2, b2, W3, b3):
    src = edge_index[0].astype(jnp.int32)
    dst = edge_index[1].astype(jnp.int32)
    padv = jnp.full((EP - E,), N, dtype=jnp.int32)
    src_p = jnp.concatenate([src, padv]).reshape(NW, NCHUNK, CHUNK)
    dst_p = jnp.concatenate([dst, padv]).reshape(NW, NCHUNK, CHUNK)
    x = jnp.zeros((NP, F), jnp.float32).at[:N].set(features)
    w3p = jnp.zeros((F, F), jnp.float32).at[:, : b3.shape[0]].set(W3)
    b3p = jnp.zeros((1, F), jnp.float32).at[0, : b3.shape[0]].set(b3)
    ones128 = jnp.ones((CHUNK, F), jnp.float32)
    zeros128 = jnp.zeros((STRIPE, F), jnp.float32)

    deg = _deg_kernel(src_p, dst_p, ones128, zeros128)
    dsrc = deg[0, :, :1]
    ddst = deg[1, :, :1]
    ns, nd = _norms(dsrc, ddst)

    t1 = _first_table(x, W1, ns)
    agg1 = _mp_kernel(t1, src_p, dst_p, zeros128)
    t2 = _mid_table(agg1, nd, b1.reshape(1, F), W2, ns)
    agg2 = _mp_kernel(t2, src_p, dst_p, zeros128)
    t3 = _mid_table(agg2, nd, b2.reshape(1, F), w3p, ns)
    agg3 = _mp_kernel(t3, src_p, dst_p, zeros128)

    out = _final(agg3, nd, b3p)
    return out[:N, : b3.shape[0]]
